# baseline (speedup 1.0000x reference)
import functools

import jax
import jax.numpy as jnp
from jax.experimental import pallas as pl
from jax.experimental.pallas import tpu as pltpu

_K = 16
_NEG_INF = float("-inf")


def _upw_body(x_ref, pos_ref, w1a_ref, w1b_ref, u_ref, pw_ref):
    w = w1b_ref[...]
    p = pos_ref[...]
    pw = p[:, 0:1] * w[0:1, :] + p[:, 1:2] * w[1:2, :] + p[:, 2:3] * w[2:3, :]
    xw = jnp.dot(x_ref[...], w1a_ref[...], preferred_element_type=jnp.float32)
    pw_ref[...] = pw
    u_ref[...] = xw + pw


def _compute_u_pw(xp, posp, w1a, w1b):
    npad, d = xp.shape
    return pl.pallas_call(
        _upw_body,
        out_shape=(
            jax.ShapeDtypeStruct((npad, d), jnp.float32),
            jax.ShapeDtypeStruct((npad, d), jnp.float32),
        ),
    )(xp, posp, w1a, w1b)


def _topk_body(posq_ref, bq_ref, pos_ref, bcol_ref, idx_ref, *, npad):
    pq = posq_ref[...]
    p = pos_ref[...]
    dx = p[:, 0:1] - pq[0:1, :]
    dy = p[:, 1:2] - pq[1:2, :]
    dz = p[:, 2:3] - pq[2:3, :]
    d = (dx * dx + dy * dy) + dz * dz
    valid = bcol_ref[...] == bq_ref[...]
    d = jnp.where(valid, d, jnp.inf)
    di = jax.lax.bitcast_convert_type(d, jnp.int32)
    cb = d.shape[1]
    ridx = jax.lax.broadcasted_iota(jnp.int32, (npad, cb), 0)
    imax = jnp.int32(2**31 - 1)
    for t in range(_K):
        m = jnp.min(di, axis=0, keepdims=True)
        cand = jnp.where(di == m, ridx, npad)
        sel = jnp.min(cand, axis=0, keepdims=True)
        idx_ref[t : t + 1, :] = sel
        di = jnp.where(ridx == sel, imax, di)


def _knn_topk(posp, batchp):
    npad = posp.shape[0]
    cb = 128
    post = posp.T
    brow = batchp.reshape(1, npad)
    bcol = batchp.reshape(npad, 1)
    return pl.pallas_call(
        functools.partial(_topk_body, npad=npad),
        grid=(npad // cb,),
        in_specs=[
            pl.BlockSpec((3, cb), lambda i: (0, i)),
            pl.BlockSpec((1, cb), lambda i: (0, i)),
            pl.BlockSpec((npad, 3), lambda i: (0, 0)),
            pl.BlockSpec((npad, 1), lambda i: (0, 0)),
        ],
        out_specs=pl.BlockSpec((_K, cb), lambda i: (0, i)),
        out_shape=jax.ShapeDtypeStruct((_K, npad), jnp.int32),
    )(post, brow, posp, bcol)


def _scatter_body(idx_ref, u_ref, m_ref, m_scr, *, nsteps, rows):
    step = pl.program_id(0)

    @pl.when(step == 0)
    def _init():
        m_scr[...] = jnp.full(m_scr.shape, _NEG_INF, jnp.float32)

    def body(j, _):
        u_row = u_ref[pl.ds(j, 1), :]
        for k in range(_K):
            i = idx_ref[k, j]
            m_scr[pl.ds(i, 1), :] = jnp.maximum(m_scr[pl.ds(i, 1), :], u_row)
        return 0

    jax.lax.fori_loop(0, rows, body, 0)

    @pl.when(step == nsteps - 1)
    def _flush():
        m_ref[...] = m_scr[...]


def _scatter_max_tc(idx, u):
    npad, d = u.shape
    rows = 128 if npad % 128 == 0 else npad
    nsteps = npad // rows
    return pl.pallas_call(
        functools.partial(_scatter_body, nsteps=nsteps, rows=rows),
        grid=(nsteps,),
        in_specs=[
            pl.BlockSpec((_K, rows), lambda i: (0, i), memory_space=pltpu.SMEM),
            pl.BlockSpec((rows, d), lambda i: (i, 0)),
        ],
        out_specs=pl.BlockSpec((npad, d), lambda i: (0, 0)),
        out_shape=jax.ShapeDtypeStruct((npad, d), jnp.float32),
        scratch_shapes=[pltpu.VMEM((npad, d), jnp.float32)],
    )(idx, u)


def _out_body(m_ref, pw_ref, b1_ref, w2_ref, b2_ref, out_ref):
    agg = jnp.maximum(m_ref[...] - pw_ref[...] + b1_ref[...], 0.0)
    out_ref[...] = (
        jnp.dot(agg, w2_ref[...], preferred_element_type=jnp.float32)
        + b2_ref[...]
    )


def _final(m, pw, b1, w2, b2):
    npad, d = m.shape
    return pl.pallas_call(
        _out_body,
        out_shape=jax.ShapeDtypeStruct((npad, d), jnp.float32),
    )(m, pw, b1.reshape(1, d), w2, b2.reshape(1, d))


def kernel(x, pos, batch, W1, b1, W2, b2):
    n, d = x.shape
    npad = ((n + 127) // 128) * 128
    pad = npad - n
    xp = jnp.pad(x, ((0, pad), (0, 0)))
    posp = jnp.pad(pos, ((0, pad), (0, 0)))
    batchp = jnp.pad(batch, (0, pad), constant_values=-1)
    w1a = W1[:d]
    w1b = W1[d:]

    u, pw = _compute_u_pw(xp, posp, w1a, w1b)
    idx = _knn_topk(posp, batchp)
    m = _scatter_max_tc(idx, u)
    out = _final(m, pw, b1, W2, b2)
    return (out[:n], pos, batch)


# speedup vs baseline: 1.0317x; 1.0317x over previous
import functools

import jax
import jax.numpy as jnp
from jax.experimental import pallas as pl
from jax.experimental.pallas import tpu as pltpu

_K = 16
_NEG_INF = float("-inf")


def _upw_body(x_ref, pos_ref, w1a_ref, w1b_ref, u_ref, pw_ref):
    w = w1b_ref[...]
    p = pos_ref[...]
    pw = p[:, 0:1] * w[0:1, :] + p[:, 1:2] * w[1:2, :] + p[:, 2:3] * w[2:3, :]
    xw = jnp.dot(x_ref[...], w1a_ref[...], preferred_element_type=jnp.float32)
    pw_ref[...] = pw
    u_ref[...] = xw + pw


def _compute_u_pw(xp, posp, w1a, w1b):
    npad, d = xp.shape
    return pl.pallas_call(
        _upw_body,
        out_shape=(
            jax.ShapeDtypeStruct((npad, d), jnp.float32),
            jax.ShapeDtypeStruct((npad, d), jnp.float32),
        ),
    )(xp, posp, w1a, w1b)


def _topk_body(posq_ref, bq_ref, pos_ref, bcol_ref, idx_ref, *, npad):
    pq = posq_ref[...]
    p = pos_ref[...]
    dx = p[:, 0:1] - pq[0:1, :]
    dy = p[:, 1:2] - pq[1:2, :]
    dz = p[:, 2:3] - pq[2:3, :]
    d = (dx * dx + dy * dy) + dz * dz
    valid = bcol_ref[...] == bq_ref[...]
    d = jnp.where(valid, d, jnp.inf)
    di = jax.lax.bitcast_convert_type(d, jnp.int32)
    cb = d.shape[1]
    g = npad // 128
    di = di.reshape(g, 128, cb)
    ridx = (
        jax.lax.broadcasted_iota(jnp.int32, (g, 128, cb), 0) * 128
        + jax.lax.broadcasted_iota(jnp.int32, (g, 128, cb), 1)
    )
    imax = jnp.int32(2**31 - 1)
    for t in range(_K):
        m = jnp.min(jnp.min(di, axis=0), axis=0, keepdims=True)
        cand = jnp.where(di == m[None], ridx, npad)
        sel = jnp.min(
            jnp.min(cand, axis=0), axis=0, keepdims=True
        )
        idx_ref[t : t + 1, :] = sel
        di = jnp.where(ridx == sel[None], imax, di)


def _knn_topk(posp, batchp):
    npad = posp.shape[0]
    cb = 128
    post = posp.T
    brow = batchp.reshape(1, npad)
    bcol = batchp.reshape(npad, 1)
    return pl.pallas_call(
        functools.partial(_topk_body, npad=npad),
        grid=(npad // cb,),
        in_specs=[
            pl.BlockSpec((3, cb), lambda i: (0, i)),
            pl.BlockSpec((1, cb), lambda i: (0, i)),
            pl.BlockSpec((npad, 3), lambda i: (0, 0)),
            pl.BlockSpec((npad, 1), lambda i: (0, 0)),
        ],
        out_specs=pl.BlockSpec((_K, cb), lambda i: (0, i)),
        out_shape=jax.ShapeDtypeStruct((_K, npad), jnp.int32),
    )(post, brow, posp, bcol)


def _scatter_body(idx_ref, u_ref, m_ref, m_scr, *, nsteps, rows):
    step = pl.program_id(0)

    @pl.when(step == 0)
    def _init():
        m_scr[...] = jnp.full(m_scr.shape, _NEG_INF, jnp.float32)

    def body(j, _):
        u_row = u_ref[pl.ds(j, 1), :]
        for k in range(_K):
            i = idx_ref[k, j]
            m_scr[pl.ds(i, 1), :] = jnp.maximum(m_scr[pl.ds(i, 1), :], u_row)
        return 0

    jax.lax.fori_loop(0, rows, body, 0)

    @pl.when(step == nsteps - 1)
    def _flush():
        m_ref[...] = m_scr[...]


def _scatter_max_tc(idx, u):
    npad, d = u.shape
    rows = 128 if npad % 128 == 0 else npad
    nsteps = npad // rows
    return pl.pallas_call(
        functools.partial(_scatter_body, nsteps=nsteps, rows=rows),
        grid=(nsteps,),
        in_specs=[
            pl.BlockSpec((_K, rows), lambda i: (0, i), memory_space=pltpu.SMEM),
            pl.BlockSpec((rows, d), lambda i: (i, 0)),
        ],
        out_specs=pl.BlockSpec((npad, d), lambda i: (0, 0)),
        out_shape=jax.ShapeDtypeStruct((npad, d), jnp.float32),
        scratch_shapes=[pltpu.VMEM((npad, d), jnp.float32)],
    )(idx, u)


def _out_body(m_ref, pw_ref, b1_ref, w2_ref, b2_ref, out_ref):
    agg = jnp.maximum(m_ref[...] - pw_ref[...] + b1_ref[...], 0.0)
    out_ref[...] = (
        jnp.dot(agg, w2_ref[...], preferred_element_type=jnp.float32)
        + b2_ref[...]
    )


def _final(m, pw, b1, w2, b2):
    npad, d = m.shape
    return pl.pallas_call(
        _out_body,
        out_shape=jax.ShapeDtypeStruct((npad, d), jnp.float32),
    )(m, pw, b1.reshape(1, d), w2, b2.reshape(1, d))


def kernel(x, pos, batch, W1, b1, W2, b2):
    n, d = x.shape
    npad = ((n + 127) // 128) * 128
    pad = npad - n
    xp = jnp.pad(x, ((0, pad), (0, 0)))
    posp = jnp.pad(pos, ((0, pad), (0, 0)))
    batchp = jnp.pad(batch, (0, pad), constant_values=-1)
    w1a = W1[:d]
    w1b = W1[d:]

    u, pw = _compute_u_pw(xp, posp, w1a, w1b)
    idx = _knn_topk(posp, batchp)
    m = _scatter_max_tc(idx, u)
    out = _final(m, pw, b1, W2, b2)
    return (out[:n], pos, batch)


# speedup vs baseline: 1.1572x; 1.1217x over previous
import functools

import jax
import jax.numpy as jnp
from jax.experimental import pallas as pl
from jax.experimental.pallas import tpu as pltpu

_K = 16
_NEG_INF = float("-inf")


def _upw_body(x_ref, pos_ref, w1a_ref, w1b_ref, u_ref, pw_ref):
    w = w1b_ref[...]
    p = pos_ref[...]
    pw = p[:, 0:1] * w[0:1, :] + p[:, 1:2] * w[1:2, :] + p[:, 2:3] * w[2:3, :]
    xw = jnp.dot(x_ref[...], w1a_ref[...], preferred_element_type=jnp.float32)
    pw_ref[...] = pw
    u_ref[...] = xw + pw


def _compute_u_pw(xp, posp, w1a, w1b):
    npad, d = xp.shape
    return pl.pallas_call(
        _upw_body,
        out_shape=(
            jax.ShapeDtypeStruct((npad, d), jnp.float32),
            jax.ShapeDtypeStruct((npad, d), jnp.float32),
        ),
    )(xp, posp, w1a, w1b)


def _topk_body(posq_ref, bq_ref, post_ref, brow_ref, idx_ref, *, npad):
    pq = posq_ref[...]
    pt = post_ref[...]
    dx = pq[:, 0:1] - pt[0:1, :]
    dy = pq[:, 1:2] - pt[1:2, :]
    dz = pq[:, 2:3] - pt[2:3, :]
    d = (dx * dx + dy * dy) + dz * dz
    valid = bq_ref[...] == brow_ref[...]
    d = jnp.where(valid, d, jnp.inf)
    di = jax.lax.bitcast_convert_type(d, jnp.int32)
    cb = d.shape[0]
    cidx = jax.lax.broadcasted_iota(jnp.int32, (cb, npad), 1)
    imax = jnp.int32(2**31 - 1)
    for t in range(_K):
        m = jnp.min(di, axis=1, keepdims=True)
        cand = jnp.where(di == m, cidx, npad)
        sel = jnp.min(cand, axis=1, keepdims=True)
        idx_ref[:, t : t + 1] = sel
        di = jnp.where(cidx == sel, imax, di)


def _knn_topk(posp, batchp):
    npad = posp.shape[0]
    cb = 128
    post = posp.T
    bcol = batchp.reshape(npad, 1)
    brow = batchp.reshape(1, npad)
    return pl.pallas_call(
        functools.partial(_topk_body, npad=npad),
        grid=(npad // cb,),
        in_specs=[
            pl.BlockSpec((cb, 3), lambda i: (i, 0)),
            pl.BlockSpec((cb, 1), lambda i: (i, 0)),
            pl.BlockSpec((3, npad), lambda i: (0, 0)),
            pl.BlockSpec((1, npad), lambda i: (0, 0)),
        ],
        out_specs=pl.BlockSpec((cb, _K), lambda i: (i, 0)),
        out_shape=jax.ShapeDtypeStruct((npad, _K), jnp.int32),
    )(posp, bcol, post, brow)


def _scatter_body(idx_ref, u_ref, m_ref, m_scr, *, nsteps, rows):
    step = pl.program_id(0)

    @pl.when(step == 0)
    def _init():
        m_scr[...] = jnp.full(m_scr.shape, _NEG_INF, jnp.float32)

    def body(j, _):
        u_row = u_ref[pl.ds(j, 1), :]
        for k in range(_K):
            i = idx_ref[j, k]
            m_scr[pl.ds(i, 1), :] = jnp.maximum(m_scr[pl.ds(i, 1), :], u_row)
        return 0

    jax.lax.fori_loop(0, rows, body, 0)

    @pl.when(step == nsteps - 1)
    def _flush():
        m_ref[...] = m_scr[...]


def _scatter_max_tc(idx, u):
    npad, d = u.shape
    rows = 632 if npad % 632 == 0 else npad
    nsteps = npad // rows
    return pl.pallas_call(
        functools.partial(_scatter_body, nsteps=nsteps, rows=rows),
        grid=(nsteps,),
        in_specs=[
            pl.BlockSpec((rows, _K), lambda i: (i, 0), memory_space=pltpu.SMEM),
            pl.BlockSpec((rows, d), lambda i: (i, 0)),
        ],
        out_specs=pl.BlockSpec((npad, d), lambda i: (0, 0)),
        out_shape=jax.ShapeDtypeStruct((npad, d), jnp.float32),
        scratch_shapes=[pltpu.VMEM((npad, d), jnp.float32)],
    )(idx, u)


def _out_body(m_ref, pw_ref, b1_ref, w2_ref, b2_ref, out_ref):
    agg = jnp.maximum(m_ref[...] - pw_ref[...] + b1_ref[...], 0.0)
    out_ref[...] = (
        jnp.dot(agg, w2_ref[...], preferred_element_type=jnp.float32)
        + b2_ref[...]
    )


def _final(m, pw, b1, w2, b2):
    npad, d = m.shape
    return pl.pallas_call(
        _out_body,
        out_shape=jax.ShapeDtypeStruct((npad, d), jnp.float32),
    )(m, pw, b1.reshape(1, d), w2, b2.reshape(1, d))


def kernel(x, pos, batch, W1, b1, W2, b2):
    n, d = x.shape
    npad = ((n + 127) // 128) * 128
    pad = npad - n
    xp = jnp.pad(x, ((0, pad), (0, 0)))
    posp = jnp.pad(pos, ((0, pad), (0, 0)))
    batchp = jnp.pad(batch, (0, pad), constant_values=-1)
    w1a = W1[:d]
    w1b = W1[d:]

    u, pw = _compute_u_pw(xp, posp, w1a, w1b)
    idx = _knn_topk(posp, batchp)
    m = _scatter_max_tc(idx, u)
    out = _final(m, pw, b1, W2, b2)
    return (out[:n], pos, batch)
